# Initial kernel scaffold; baseline (speedup 1.0000x reference)
import jax
import jax.numpy as jnp
from jax.experimental import pallas as pl


def kernel(x, tss_indexes, pe):
    raise NotImplementedError("write your pallas kernel here")


# speedup vs baseline: 3.3015x; 3.3015x over previous
import functools

import jax
import jax.numpy as jnp
from jax import lax
from jax.experimental import pallas as pl
from jax.experimental.pallas import tpu as pltpu
from jax.experimental.pallas import tpu_sc as plsc

B, L, D = 1024, 200, 128
N = B * L
NC, NS = 2, 16
NW = NC * NS
PER_W = N // NW
CH = 128
NCHUNK = PER_W // CH
LANES = 16


def _pe_add_body(x_hbm, idx_hbm, pe_hbm, out_hbm, idx_v, xb_v, rows_v, sem):
    wid = lax.axis_index("s") * NC + lax.axis_index("c")
    base = wid * PER_W

    def chunk(ci, _):
        off = base + ci * CH
        pltpu.sync_copy(idx_hbm.at[pl.ds(off, CH)], idx_v)
        gather = pltpu.async_copy(pe_hbm.at[idx_v], rows_v, sem)
        pltpu.sync_copy(x_hbm.at[pl.ds(off, CH)], xb_v)
        gather.wait()

        def add_row(r, _):
            for c in range(D // LANES):
                sl = pl.ds(c * LANES, LANES)
                plsc.addupdate(xb_v.at[r, sl], rows_v[r, sl])
            return ()

        lax.fori_loop(0, CH, add_row, ())
        pltpu.sync_copy(xb_v, out_hbm.at[pl.ds(off, CH)])
        return ()

    lax.fori_loop(0, NCHUNK, chunk, ())


@functools.partial(jax.jit, static_argnames=())
def kernel(x, tss_indexes, pe):
    xf = x.reshape(N, D)
    idx = tss_indexes.reshape(N).astype(jnp.int32)
    mesh = plsc.VectorSubcoreMesh(
        core_axis_name="c", subcore_axis_name="s",
        num_cores=NC, num_subcores=NS,
    )
    out = pl.kernel(
        _pe_add_body,
        out_type=jax.ShapeDtypeStruct((N, D), jnp.float32),
        mesh=mesh,
        scratch_types=[
            pltpu.VMEM((CH,), jnp.int32),
            pltpu.VMEM((CH, D), jnp.float32),
            pltpu.VMEM((CH, D), jnp.float32),
            pltpu.SemaphoreType.DMA,
        ],
    )(xf, idx, pe)
    return out.reshape(B, L, D)


# speedup vs baseline: 5.7884x; 1.7533x over previous
import functools

import jax
import jax.numpy as jnp
from jax import lax
from jax.experimental import pallas as pl
from jax.experimental.pallas import tpu as pltpu
from jax.experimental.pallas import tpu_sc as plsc

B, L, D = 1024, 200, 128
N = B * L
NC, NS = 2, 16
NW = NC * NS
PER_W = N // NW
CH = 128
NCHUNK = PER_W // CH
LANES = 16


def _pe_add_body(x_hbm, idx_hbm, pe_hbm, out_hbm,
                 idx0, idx1, xb0, xb1, rows0, rows1, ob0, ob1,
                 gsem, osem):
    wid = lax.axis_index("s") * NC + lax.axis_index("c")
    base = wid * PER_W
    bufs = ((idx0, xb0, rows0, ob0), (idx1, xb1, rows1, ob1))

    def fire(ci, b):
        idx_v, xb_v, rows_v, _ = bufs[b]
        off = base + ci * CH
        pltpu.sync_copy(idx_hbm.at[pl.ds(off, CH)], idx_v)
        pltpu.async_copy(x_hbm.at[pl.ds(off, CH)], xb_v, gsem)
        pltpu.async_copy(pe_hbm.at[idx_v], rows_v, gsem)

    def process(ci, b, wait_out):
        idx_v, xb_v, rows_v, ob_v = bufs[b]
        off = base + ci * CH
        pltpu.make_async_copy(x_hbm.at[pl.ds(off, CH)], xb_v, gsem).wait()
        pltpu.make_async_copy(pe_hbm.at[idx_v], rows_v, gsem).wait()
        if wait_out:
            pltpu.make_async_copy(ob_v, out_hbm.at[pl.ds(off, CH)], osem).wait()

        @plsc.parallel_loop(0, CH, unroll=2)
        def add_row(r):
            for c in range(D // LANES):
                sl = pl.ds(c * LANES, LANES)
                ob_v[r, sl] = xb_v[r, sl] + rows_v[r, sl]

        pltpu.async_copy(ob_v, out_hbm.at[pl.ds(off, CH)], osem)

    fire(0, 0)
    fire(1, 1)
    process(0, 0, wait_out=False)
    fire(2, 0)
    process(1, 1, wait_out=False)
    fire(3, 1)

    def pair(q, _):
        c0 = 2 * q
        process(c0, 0, wait_out=True)
        fire(c0 + 2, 0)
        process(c0 + 1, 1, wait_out=True)
        fire(c0 + 3, 1)
        return ()

    lax.fori_loop(1, (NCHUNK - 2) // 2, pair, ())
    process(NCHUNK - 2, 0, wait_out=True)
    process(NCHUNK - 1, 1, wait_out=True)
    pltpu.make_async_copy(
        ob0, out_hbm.at[pl.ds(base + (NCHUNK - 2) * CH, CH)], osem).wait()
    pltpu.make_async_copy(
        ob1, out_hbm.at[pl.ds(base + (NCHUNK - 1) * CH, CH)], osem).wait()


@jax.jit
def kernel(x, tss_indexes, pe):
    xf = x.reshape(N, D)
    idx = tss_indexes.reshape(N).astype(jnp.int32)
    mesh = plsc.VectorSubcoreMesh(
        core_axis_name="c", subcore_axis_name="s",
        num_cores=NC, num_subcores=NS,
    )
    out = pl.kernel(
        _pe_add_body,
        out_type=jax.ShapeDtypeStruct((N, D), jnp.float32),
        mesh=mesh,
        scratch_types=[
            pltpu.VMEM((CH,), jnp.int32),
            pltpu.VMEM((CH,), jnp.int32),
            pltpu.VMEM((CH, D), jnp.float32),
            pltpu.VMEM((CH, D), jnp.float32),
            pltpu.VMEM((CH, D), jnp.float32),
            pltpu.VMEM((CH, D), jnp.float32),
            pltpu.VMEM((CH, D), jnp.float32),
            pltpu.VMEM((CH, D), jnp.float32),
            pltpu.SemaphoreType.DMA,
            pltpu.SemaphoreType.DMA,
        ],
    )(xf, idx, pe)
    return out.reshape(B, L, D)
